# baseline (speedup 1.0000x reference)
import math

import numpy as np
import jax
import jax.numpy as jnp
from jax import lax
from jax.experimental import pallas as pl
from jax.experimental.pallas import tpu as pltpu
from jax.experimental.pallas import tpu_sc as plsc

D_MODEL = 768
N_HEADS = 12
D_FF = 2048
E_LAYERS = 3
FACTOR = 5
B, L = 2, 2048
DH = D_MODEL // N_HEADS
U = min(int(FACTOR * math.ceil(math.log(L))), L)

NEG = -1e30


def _build_counts():
    try:
        counts = []
        with jax.default_device(jax.devices("cpu")[0]):
            for i in range(E_LAYERS):
                k = jax.random.fold_in(jax.random.key(42), i)
                idx = np.asarray(jax.random.randint(k, (L, U), 0, L))
                c = np.zeros((L, L), np.int8)
                np.add.at(c, (np.arange(L)[:, None], idx), 1)
                counts.append(c)
        return counts
    except Exception:
        return None


_COUNTS = _build_counts()


def _count_matrix(i):
    if _COUNTS is not None:
        return jnp.asarray(_COUNTS[i])
    k = jax.random.fold_in(jax.random.key(42), i)
    idx = jax.random.randint(k, (L, U), 0, L)
    return jnp.zeros((L, L), jnp.int8).at[
        jnp.arange(L)[:, None], idx].add(jnp.int8(1))

R_QKV = 512


def _qkv_body(x_ref, w_ref, b_ref, q_ref, k_ref, v_ref):
    x = x_ref[0]
    y = lax.dot_general(x, w_ref[...], (((1,), (1,)), ((), ())),
                        preferred_element_type=jnp.float32)
    y = y + b_ref[...]
    for h in range(N_HEADS):
        q_ref[0, h] = y[:, h * DH:(h + 1) * DH]
        k_ref[0, h] = y[:, D_MODEL + h * DH:D_MODEL + (h + 1) * DH]
        v_ref[0, h] = y[:, 2 * D_MODEL + h * DH:2 * D_MODEL + (h + 1) * DH]


def _qkv(x, w_qkv, b_qkv):
    out = jax.ShapeDtypeStruct((B, N_HEADS, L, DH), jnp.float32)
    hspec = pl.BlockSpec((1, N_HEADS, R_QKV, DH), lambda b, l: (b, 0, l, 0))
    return pl.pallas_call(
        _qkv_body,
        grid=(B, L // R_QKV),
        in_specs=[
            pl.BlockSpec((1, R_QKV, D_MODEL), lambda b, l: (b, l, 0)),
            pl.BlockSpec((3 * D_MODEL, D_MODEL), lambda b, l: (0, 0)),
            pl.BlockSpec((1, 3 * D_MODEL), lambda b, l: (0, 0)),
        ],
        out_specs=[hspec, hspec, hspec],
        out_shape=[out, out, out],
    )(x, w_qkv, b_qkv)


R_M = 256


def _m_body(q_ref, k_ref, c_ref, m_ref):
    c = c_ref[...].astype(jnp.float32)
    sampled = c > 0.0
    cols = []
    for h in range(N_HEADS):
        q_h = q_ref[0, h]
        k_h = k_ref[0, h]
        s = lax.dot_general(q_h, k_h, (((1,), (1,)), ((), ())),
                            preferred_element_type=jnp.float32)
        smax = jnp.max(jnp.where(sampled, s, NEG), axis=1, keepdims=True)
        ssum = jnp.sum(s * c, axis=1, keepdims=True)
        cols.append(smax - ssum * (1.0 / L))
    m_ref[0] = jnp.concatenate(cols, axis=1)


def _measure_m(q, k, c):
    return pl.pallas_call(
        _m_body,
        grid=(B, L // R_M),
        in_specs=[
            pl.BlockSpec((1, N_HEADS, R_M, DH), lambda b, l: (b, 0, l, 0)),
            pl.BlockSpec((1, N_HEADS, L, DH), lambda b, l: (b, 0, 0, 0)),
            pl.BlockSpec((R_M, L), lambda b, l: (l, 0)),
        ],
        out_specs=pl.BlockSpec((1, R_M, N_HEADS), lambda b, l: (b, l, 0)),
        out_shape=jax.ShapeDtypeStruct((B, L, N_HEADS), jnp.float32),
    )(q, k, c)


def _topk_body(m_ref, o_ref):
    m = m_ref[...]
    iota = lax.broadcasted_iota(jnp.int32, (B * N_HEADS, L), 1)
    cols = []
    for _ in range(U):
        mx = jnp.max(m, axis=1, keepdims=True)
        eq = m >= mx
        idx = jnp.min(jnp.where(eq, iota, L), axis=1, keepdims=True)
        cols.append(idx)
        m = jnp.where(iota == idx, NEG, m)
    o_ref[...] = jnp.concatenate(cols, axis=1)


def _topk(m_bhl):
    return pl.pallas_call(
        _topk_body,
        in_specs=[pl.BlockSpec((B * N_HEADS, L), lambda: (0, 0))],
        out_specs=pl.BlockSpec((B * N_HEADS, U), lambda: (0, 0)),
        out_shape=jax.ShapeDtypeStruct((B * N_HEADS, U), jnp.int32),
    )(m_bhl)


def _attn_body(idx_ref, q_ref, k_ref, v_ref, ctx_ref, mv_ref, qr):
    for j in range(U):
        row = idx_ref[0, 0, j]
        qr[pl.ds(j, 1), :] = q_ref[0, 0, pl.ds(row, 1), :]
    k = k_ref[0, 0]
    v = v_ref[0, 0]
    s = lax.dot_general(qr[...], k, (((1,), (1,)), ((), ())),
                        preferred_element_type=jnp.float32)
    s = s * (1.0 / math.sqrt(DH))
    s = s - jnp.max(s, axis=1, keepdims=True)
    e = jnp.exp(s)
    a = e / jnp.sum(e, axis=1, keepdims=True)
    ctx_ref[0, 0] = jnp.dot(a, v, preferred_element_type=jnp.float32)
    mv_ref[0, 0] = jnp.mean(v, axis=0, keepdims=True)


def _attention(m_top, q, k, v):
    return pl.pallas_call(
        _attn_body,
        grid=(B, N_HEADS),
        in_specs=[
            pl.BlockSpec((1, 1, U), lambda b, h: (b * N_HEADS + h, 0, 0),
                         memory_space=pltpu.SMEM),
            pl.BlockSpec((1, 1, L, DH), lambda b, h: (b, h, 0, 0)),
            pl.BlockSpec((1, 1, L, DH), lambda b, h: (b, h, 0, 0)),
            pl.BlockSpec((1, 1, L, DH), lambda b, h: (b, h, 0, 0)),
        ],
        out_specs=[
            pl.BlockSpec((1, 1, U, DH), lambda b, h: (b, h, 0, 0)),
            pl.BlockSpec((1, 1, 1, DH), lambda b, h: (b, h, 0, 0)),
        ],
        out_shape=[
            jax.ShapeDtypeStruct((B, N_HEADS, U, DH), jnp.float32),
            jax.ShapeDtypeStruct((B, N_HEADS, 1, DH), jnp.float32),
        ],
        scratch_shapes=[pltpu.VMEM((U, DH), jnp.float32)],
    )(m_top, q, k, v)


_NC, _NS = 2, 16
_FILLB = 512


def _sc_assemble_body(mtop_hbm, cu_hbm, mv_hbm, ctx_hbm,
                      mvv, idxv, rows, fillbuf, sem):
    w = lax.axis_index("s") * _NC + lax.axis_index("c")

    @pl.when(w < B * N_HEADS)
    def _():
        pltpu.sync_copy(mv_hbm.at[w], mvv)
        chunks = [mvv[pl.ds(c * 16, 16)] for c in range(DH // 16)]

        def fill_row(j, carry):
            for c in range(DH // 16):
                fillbuf[j, pl.ds(c * 16, 16)] = chunks[c]
            return carry

        lax.fori_loop(0, _FILLB, fill_row, 0)
        for d in range(L // _FILLB):
            pltpu.sync_copy(fillbuf,
                            ctx_hbm.at[w, pl.ds(d * _FILLB, _FILLB)])
        pltpu.sync_copy(mtop_hbm.at[w], idxv)
        pltpu.sync_copy(cu_hbm.at[w], rows)
        pltpu.async_copy(rows, ctx_hbm.at[w].at[idxv], sem).wait()


def _assemble(m_top, ctx_upd, mean_v):
    f = pl.kernel(
        _sc_assemble_body,
        out_type=jax.ShapeDtypeStruct((B * N_HEADS, L, DH), jnp.float32),
        mesh=plsc.VectorSubcoreMesh(core_axis_name="c", subcore_axis_name="s"),
        compiler_params=pltpu.CompilerParams(use_tc_tiling_on_sc=False),
        scratch_types=[
            pltpu.VMEM((DH,), jnp.float32),
            pltpu.VMEM((U,), jnp.int32),
            pltpu.VMEM((U, DH), jnp.float32),
            pltpu.VMEM((_FILLB, DH), jnp.float32),
            pltpu.SemaphoreType.DMA,
        ],
    )
    ctx = f(m_top.reshape(B * N_HEADS, U),
            ctx_upd.reshape(B * N_HEADS, U, DH),
            mean_v.reshape(B * N_HEADS, DH))
    return ctx.reshape(B, N_HEADS, L, DH)


R_T = 256


def _layer_norm(t, g, b):
    mu = jnp.mean(t, axis=1, keepdims=True)
    var = jnp.mean((t - mu) ** 2, axis=1, keepdims=True)
    return (t - mu) / jnp.sqrt(var + 1e-5) * g + b


def _tail_body(ctx_ref, x_ref, wo_ref, bo_ref, c1w_ref, c1b_ref,
               c2w_ref, c2b_ref, g1_ref, b1_ref, g2_ref, b2_ref,
               o_ref, cat):
    for h in range(N_HEADS):
        cat[:, h * DH:(h + 1) * DH] = ctx_ref[0, h]
    proj = lax.dot_general(cat[...], wo_ref[...], (((1,), (1,)), ((), ())),
                           preferred_element_type=jnp.float32)
    t = x_ref[0] + proj + bo_ref[...]
    x1 = _layer_norm(t, g1_ref[...], b1_ref[...])
    y = lax.dot_general(x1, c1w_ref[...], (((1,), (1,)), ((), ())),
                        preferred_element_type=jnp.float32)
    y = jnp.maximum(y + c1b_ref[...], 0.0)
    y = lax.dot_general(y, c2w_ref[...], (((1,), (1,)), ((), ())),
                        preferred_element_type=jnp.float32)
    y = y + c2b_ref[...]
    o_ref[0] = _layer_norm(x1 + y, g2_ref[...], b2_ref[...])


def _tail(ctx, x, wo, bo, c1w, c1b, c2w, c2b, g1, b1, g2, b2):
    full = lambda shape: pl.BlockSpec(shape, lambda b, l: tuple(0 for _ in shape))
    return pl.pallas_call(
        _tail_body,
        grid=(B, L // R_T),
        in_specs=[
            pl.BlockSpec((1, N_HEADS, R_T, DH), lambda b, l: (b, 0, l, 0)),
            pl.BlockSpec((1, R_T, D_MODEL), lambda b, l: (b, l, 0)),
            full((D_MODEL, D_MODEL)),
            full((1, D_MODEL)),
            full((D_FF, D_MODEL)),
            full((1, D_FF)),
            full((D_MODEL, D_FF)),
            full((1, D_MODEL)),
            full((1, D_MODEL)),
            full((1, D_MODEL)),
            full((1, D_MODEL)),
            full((1, D_MODEL)),
        ],
        out_specs=pl.BlockSpec((1, R_T, D_MODEL), lambda b, l: (b, l, 0)),
        out_shape=jax.ShapeDtypeStruct((B, L, D_MODEL), jnp.float32),
        scratch_shapes=[pltpu.VMEM((R_T, D_MODEL), jnp.float32)],
    )(ctx, x, wo, bo, c1w, c1b, c2w, c2b, g1, b1, g2, b2)


def kernel(x_enc, Wq, bq, Wk, bk, Wv, bv, Wo, bo, c1w, c1b, c2w, c2b,
           g1, be1, g2, be2):
    x = x_enc
    for i in range(E_LAYERS):
        w_qkv = jnp.concatenate([Wq[i], Wk[i], Wv[i]], axis=0)
        b_qkv = jnp.concatenate([bq[i], bk[i], bv[i]])[None, :]
        q, k, v = _qkv(x, w_qkv, b_qkv)
        m = _measure_m(q, k, _count_matrix(i))
        m_bhl = m.transpose(0, 2, 1).reshape(B * N_HEADS, L)
        m_top = _topk(m_bhl).reshape(B * N_HEADS, 1, U)
        ctx_upd, mean_v = _attention(m_top, q, k, v)
        ctx = _assemble(m_top, ctx_upd, mean_v)
        x = _tail(ctx, x, Wo[i], bo[i][None, :], c1w[i], c1b[i][None, :],
                  c2w[i], c2b[i][None, :], g1[i][None, :], be1[i][None, :],
                  g2[i][None, :], be2[i][None, :])
    return x


# speedup vs baseline: 1.0490x; 1.0490x over previous
import math

import numpy as np
import jax
import jax.numpy as jnp
from jax import lax
from jax.experimental import pallas as pl
from jax.experimental.pallas import tpu as pltpu
from jax.experimental.pallas import tpu_sc as plsc

D_MODEL = 768
N_HEADS = 12
D_FF = 2048
E_LAYERS = 3
FACTOR = 5
B, L = 2, 2048
DH = D_MODEL // N_HEADS
U = min(int(FACTOR * math.ceil(math.log(L))), L)

NEG = -1e30


def _build_counts():
    try:
        counts = []
        with jax.default_device(jax.devices("cpu")[0]):
            for i in range(E_LAYERS):
                k = jax.random.fold_in(jax.random.key(42), i)
                idx = np.asarray(jax.random.randint(k, (L, U), 0, L))
                c = np.zeros((L, L), np.int8)
                np.add.at(c, (np.arange(L)[:, None], idx), 1)
                counts.append(c)
        return counts
    except Exception:
        return None


_COUNTS = _build_counts()


def _count_matrix(i):
    if _COUNTS is not None:
        return jnp.asarray(_COUNTS[i])
    k = jax.random.fold_in(jax.random.key(42), i)
    idx = jax.random.randint(k, (L, U), 0, L)
    return jnp.zeros((L, L), jnp.int8).at[
        jnp.arange(L)[:, None], idx].add(jnp.int8(1))

R_QKV = 512


def _qkv_body(x_ref, w_ref, b_ref, q_ref, k_ref, v_ref):
    x = x_ref[0]
    y = lax.dot_general(x, w_ref[...], (((1,), (1,)), ((), ())),
                        preferred_element_type=jnp.float32)
    y = y + b_ref[...]
    for h in range(N_HEADS):
        q_ref[0, h] = y[:, h * DH:(h + 1) * DH]
        k_ref[0, h] = y[:, D_MODEL + h * DH:D_MODEL + (h + 1) * DH]
        v_ref[0, h] = y[:, 2 * D_MODEL + h * DH:2 * D_MODEL + (h + 1) * DH]


def _qkv(x, w_qkv, b_qkv):
    out = jax.ShapeDtypeStruct((B, N_HEADS, L, DH), jnp.float32)
    hspec = pl.BlockSpec((1, N_HEADS, R_QKV, DH), lambda b, l: (b, 0, l, 0))
    return pl.pallas_call(
        _qkv_body,
        grid=(B, L // R_QKV),
        in_specs=[
            pl.BlockSpec((1, R_QKV, D_MODEL), lambda b, l: (b, l, 0)),
            pl.BlockSpec((3 * D_MODEL, D_MODEL), lambda b, l: (0, 0)),
            pl.BlockSpec((1, 3 * D_MODEL), lambda b, l: (0, 0)),
        ],
        out_specs=[hspec, hspec, hspec],
        out_shape=[out, out, out],
    )(x, w_qkv, b_qkv)


R_M = 256


def _m_body(q_ref, k_ref, c_ref, m_ref):
    c = c_ref[...].astype(jnp.float32)
    sampled = c > 0.0
    cols = []
    for h in range(N_HEADS):
        q_h = q_ref[0, h]
        k_h = k_ref[0, h]
        s = lax.dot_general(q_h, k_h, (((1,), (1,)), ((), ())),
                            preferred_element_type=jnp.float32)
        smax = jnp.max(jnp.where(sampled, s, NEG), axis=1, keepdims=True)
        ssum = jnp.sum(s * c, axis=1, keepdims=True)
        cols.append(smax - ssum * (1.0 / L))
    m_ref[0] = jnp.concatenate(cols, axis=1)


def _measure_m(q, k, c):
    return pl.pallas_call(
        _m_body,
        grid=(B, L // R_M),
        in_specs=[
            pl.BlockSpec((1, N_HEADS, R_M, DH), lambda b, l: (b, 0, l, 0)),
            pl.BlockSpec((1, N_HEADS, L, DH), lambda b, l: (b, 0, 0, 0)),
            pl.BlockSpec((R_M, L), lambda b, l: (l, 0)),
        ],
        out_specs=pl.BlockSpec((1, R_M, N_HEADS), lambda b, l: (b, l, 0)),
        out_shape=jax.ShapeDtypeStruct((B, L, N_HEADS), jnp.float32),
    )(q, k, c)


def _topk_body(m_ref, o_ref):
    m = m_ref[...]
    iota = lax.broadcasted_iota(jnp.int32, (B * N_HEADS, L), 1)
    cols = []
    for _ in range(U):
        mx = jnp.max(m, axis=1, keepdims=True)
        eq = m >= mx
        idx = jnp.min(jnp.where(eq, iota, L), axis=1, keepdims=True)
        cols.append(idx)
        m = jnp.where(iota == idx, NEG, m)
    o_ref[...] = jnp.concatenate(cols, axis=1)


def _topk(m_bhl):
    return pl.pallas_call(
        _topk_body,
        in_specs=[pl.BlockSpec((B * N_HEADS, L), lambda: (0, 0))],
        out_specs=pl.BlockSpec((B * N_HEADS, U), lambda: (0, 0)),
        out_shape=jax.ShapeDtypeStruct((B * N_HEADS, U), jnp.int32),
    )(m_bhl)


def _attn_body(idx_ref, q_ref, k_ref, v_ref, wo_ref, corr_ref, mv_ref, qr):
    for j in range(U):
        row = idx_ref[0, 0, j]
        qr[pl.ds(j, 1), :] = q_ref[0, 0, pl.ds(row, 1), :]
    k = k_ref[0, 0]
    v = v_ref[0, 0]
    s = lax.dot_general(qr[...], k, (((1,), (1,)), ((), ())),
                        preferred_element_type=jnp.float32)
    s = s * (1.0 / math.sqrt(DH))
    s = s - jnp.max(s, axis=1, keepdims=True)
    e = jnp.exp(s)
    a = e / jnp.sum(e, axis=1, keepdims=True)
    ctx = jnp.dot(a, v, preferred_element_type=jnp.float32)
    mv = jnp.mean(v, axis=0, keepdims=True)
    mv_ref[0, 0] = mv
    corr_ref[0] = jnp.dot(ctx - mv, wo_ref[...],
                          preferred_element_type=jnp.float32)


def _attention(m_top, q, k, v, wo_t):
    return pl.pallas_call(
        _attn_body,
        grid=(B, N_HEADS),
        in_specs=[
            pl.BlockSpec((1, 1, U), lambda b, h: (b * N_HEADS + h, 0, 0),
                         memory_space=pltpu.SMEM),
            pl.BlockSpec((1, 1, L, DH), lambda b, h: (b, h, 0, 0)),
            pl.BlockSpec((1, 1, L, DH), lambda b, h: (b, h, 0, 0)),
            pl.BlockSpec((1, 1, L, DH), lambda b, h: (b, h, 0, 0)),
            pl.BlockSpec((DH, D_MODEL), lambda b, h: (h, 0)),
        ],
        out_specs=[
            pl.BlockSpec((1, U, D_MODEL), lambda b, h: (b * N_HEADS + h, 0, 0)),
            pl.BlockSpec((1, 1, 1, DH), lambda b, h: (b, h, 0, 0)),
        ],
        out_shape=[
            jax.ShapeDtypeStruct((B * N_HEADS, U, D_MODEL), jnp.float32),
            jax.ShapeDtypeStruct((B, N_HEADS, 1, DH), jnp.float32),
        ],
        scratch_shapes=[pltpu.VMEM((U, DH), jnp.float32)],
    )(m_top, q, k, v, wo_t)


_UPB = N_HEADS * U


def _merge_body(idxr_ref, idxc_ref, corr_ref, out_ref):
    eq = jnp.where(idxc_ref[0] == idxr_ref[0], 1.0, 0.0)
    out_ref[0] = jnp.dot(eq, corr_ref[0], preferred_element_type=jnp.float32)


def _merge(idxr, idxc, corr):
    return pl.pallas_call(
        _merge_body,
        grid=(B,),
        in_specs=[
            pl.BlockSpec((1, 1, _UPB), lambda b: (b, 0, 0)),
            pl.BlockSpec((1, _UPB, 1), lambda b: (b, 0, 0)),
            pl.BlockSpec((1, _UPB, D_MODEL), lambda b: (b, 0, 0)),
        ],
        out_specs=pl.BlockSpec((1, _UPB, D_MODEL), lambda b: (b, 0, 0)),
        out_shape=jax.ShapeDtypeStruct((B, _UPB, D_MODEL), jnp.float32),
    )(idxr, idxc, corr)


_NTILES = 16
_ZROWS = 32
_UPT = 32


def _sc_corr_body(mtop_hbm, corr_hbm, out_hbm, idxv, rows):
    c = lax.axis_index("c")
    s = lax.axis_index("s")

    def zrow(j, carry):
        for t in range(D_MODEL // 16):
            rows[j, pl.ds(t * 16, 16)] = jnp.zeros((16,), jnp.float32)
        return carry

    lax.fori_loop(0, _ZROWS, zrow, 0)
    stripe = L // _NTILES
    for d in range(stripe // _ZROWS):
        pltpu.sync_copy(rows, out_hbm.at[c, pl.ds(s * stripe + d * _ZROWS,
                                                  _ZROWS)])
    plsc.subcore_barrier()

    @pl.when(s < _UPB // _UPT)
    def _():
        base = c * _UPB + s * _UPT
        pltpu.sync_copy(mtop_hbm.at[pl.ds(base, _UPT)], idxv)
        pltpu.sync_copy(corr_hbm.at[pl.ds(base, _UPT)], rows)
        pltpu.sync_copy(rows, out_hbm.at[c].at[idxv])


def _corr_scatter(m_top_flat, corr):
    f = pl.kernel(
        _sc_corr_body,
        out_type=jax.ShapeDtypeStruct((B, L, D_MODEL), jnp.float32),
        mesh=plsc.VectorSubcoreMesh(core_axis_name="c", subcore_axis_name="s"),
        scratch_types=[
            pltpu.VMEM((_UPT,), jnp.int32),
            pltpu.VMEM((_UPT, D_MODEL), jnp.float32),
        ],
    )
    return f(m_top_flat, corr)


R_T = 256


def _layer_norm(t, g, b):
    mu = jnp.mean(t, axis=1, keepdims=True)
    var = jnp.mean((t - mu) ** 2, axis=1, keepdims=True)
    return (t - mu) / jnp.sqrt(var + 1e-5) * g + b


def _tail_body(corr_ref, mv_ref, x_ref, wo_ref, bo_ref, c1w_ref, c1b_ref,
               c2w_ref, c2b_ref, g1_ref, b1_ref, g2_ref, b2_ref,
               o_ref):
    base = lax.dot_general(mv_ref[0], wo_ref[...], (((1,), (1,)), ((), ())),
                           preferred_element_type=jnp.float32)
    t = x_ref[0] + corr_ref[0] + base + bo_ref[...]
    x1 = _layer_norm(t, g1_ref[...], b1_ref[...])
    y = lax.dot_general(x1, c1w_ref[...], (((1,), (1,)), ((), ())),
                        preferred_element_type=jnp.float32)
    y = jnp.maximum(y + c1b_ref[...], 0.0)
    y = lax.dot_general(y, c2w_ref[...], (((1,), (1,)), ((), ())),
                        preferred_element_type=jnp.float32)
    y = y + c2b_ref[...]
    o_ref[0] = _layer_norm(x1 + y, g2_ref[...], b2_ref[...])


def _tail(corr, mv_cat, x, wo, bo, c1w, c1b, c2w, c2b, g1, b1, g2, b2):
    full = lambda shape: pl.BlockSpec(shape, lambda b, l: tuple(0 for _ in shape))
    return pl.pallas_call(
        _tail_body,
        grid=(B, L // R_T),
        in_specs=[
            pl.BlockSpec((1, R_T, D_MODEL), lambda b, l: (b, l, 0)),
            pl.BlockSpec((1, 1, D_MODEL), lambda b, l: (b, 0, 0)),
            pl.BlockSpec((1, R_T, D_MODEL), lambda b, l: (b, l, 0)),
            full((D_MODEL, D_MODEL)),
            full((1, D_MODEL)),
            full((D_FF, D_MODEL)),
            full((1, D_FF)),
            full((D_MODEL, D_FF)),
            full((1, D_MODEL)),
            full((1, D_MODEL)),
            full((1, D_MODEL)),
            full((1, D_MODEL)),
            full((1, D_MODEL)),
        ],
        out_specs=pl.BlockSpec((1, R_T, D_MODEL), lambda b, l: (b, l, 0)),
        out_shape=jax.ShapeDtypeStruct((B, L, D_MODEL), jnp.float32),
    )(corr, mv_cat, x, wo, bo, c1w, c1b, c2w, c2b, g1, b1, g2, b2)


def kernel(x_enc, Wq, bq, Wk, bk, Wv, bv, Wo, bo, c1w, c1b, c2w, c2b,
           g1, be1, g2, be2):
    x = x_enc
    for i in range(E_LAYERS):
        w_qkv = jnp.concatenate([Wq[i], Wk[i], Wv[i]], axis=0)
        b_qkv = jnp.concatenate([bq[i], bk[i], bv[i]])[None, :]
        q, k, v = _qkv(x, w_qkv, b_qkv)
        m = _measure_m(q, k, _count_matrix(i))
        m_bhl = m.transpose(0, 2, 1).reshape(B * N_HEADS, L)
        m_top = _topk(m_bhl).reshape(B * N_HEADS, 1, U)
        corr, mean_v = _attention(m_top, q, k, v, Wo[i].T)
        idx_f = m_top.reshape(B, N_HEADS * U).astype(jnp.float32)
        corr_m = _merge(idx_f.reshape(B, 1, N_HEADS * U),
                        idx_f.reshape(B, N_HEADS * U, 1),
                        corr.reshape(B, N_HEADS * U, D_MODEL))
        corr_sum = _corr_scatter(m_top.reshape(B * N_HEADS * U),
                                 corr_m.reshape(B * N_HEADS * U, D_MODEL))
        mv_cat = mean_v.reshape(B, 1, D_MODEL)
        x = _tail(corr_sum, mv_cat, x, Wo[i], bo[i][None, :],
                  c1w[i], c1b[i][None, :], c2w[i], c2b[i][None, :],
                  g1[i][None, :], be1[i][None, :],
                  g2[i][None, :], be2[i][None, :])
    return x
